# baseline (speedup 1.0000x reference)
import functools
import math

import jax
import jax.numpy as jnp
from jax import lax
from jax.experimental import pallas as pl
from jax.experimental.pallas import tpu as pltpu
from jax.experimental.pallas import tpu_sc as plsc

D_MODEL = 64
SCALE = math.sqrt(D_MODEL)

NW = 32
CH0, CH1 = 104, 96
NB = 4
LEAD = 2


def _make_kernel(Bb, S, V):
    rows_per_w = Bb // NW
    n_ch = rows_per_w * 2

    def ch_size(k):
        return CH0 if (k & 1) == 0 else CH1

    def ch_off(k):
        return 0 if (k & 1) == 0 else CH0

    mesh = plsc.VectorSubcoreMesh(core_axis_name="c", subcore_axis_name="s")

    @functools.partial(
        pl.kernel,
        mesh=mesh,
        out_type=jax.ShapeDtypeStruct((Bb, S, D_MODEL), jnp.float32),
        scratch_types=[
            pltpu.VMEM((rows_per_w, S), jnp.int32),
            [pltpu.VMEM((CH0, D_MODEL), jnp.float32) for _ in range(NB)],
            [pltpu.SemaphoreType.DMA for _ in range(NB)],
            [pltpu.SemaphoreType.DMA for _ in range(NB)],
        ],
        compiler_params=pltpu.CompilerParams(use_tc_tiling_on_sc=False),
    )
    def emb_kernel(x_hbm, lut_hbm, out_hbm, idx_v, bufs, gsems, osems):
        wid = lax.axis_index("s") * 2 + lax.axis_index("c")
        base = wid * rows_per_w
        pltpu.sync_copy(x_hbm.at[pl.ds(base, rows_per_w)], idx_v)

        def issue_gather(r, k, b):
            sz = ch_size(k)
            pltpu.async_copy(
                lut_hbm.at[idx_v.at[r, pl.ds(ch_off(k), sz)]],
                bufs[b].at[pl.ds(0, sz)], gsems[b])

        for k in range(LEAD):
            issue_gather(k >> 1, k, k)

        @pl.loop(0, n_ch, step=NB)
        def block(j0):
            r0 = j0 // 2
            for k in range(NB):
                j = j0 + k
                sz = ch_size(k)
                buf = bufs[k]
                pltpu.make_async_copy(
                    lut_hbm.at[idx_v.at[0, pl.ds(0, sz)]],
                    buf.at[pl.ds(0, sz)], gsems[k]).wait()

                @pl.loop(0, sz, unroll=2)
                def srow(r):
                    for cc in range(D_MODEL // 16):
                        sl = pl.ds(cc * 16, 16)
                        buf[r, sl] = buf[r, sl] * SCALE

                r = r0 + (k >> 1)
                pltpu.async_copy(
                    buf.at[pl.ds(0, sz)],
                    out_hbm.at[base + r, pl.ds(ch_off(k), sz)], osems[k])

                jn = j + LEAD
                kn = (k + LEAD) % NB
                szn = ch_size(kn)

                @pl.when(jn < n_ch)
                def _issue():
                    @pl.when(jn >= NB)
                    def _drain():
                        pltpu.make_async_copy(
                            bufs[kn].at[pl.ds(0, szn)],
                            out_hbm.at[base, pl.ds(0, szn)],
                            osems[kn]).wait()

                    issue_gather(r0 + ((k + LEAD) >> 1), kn, kn)

        for k in range(NB):
            sz = ch_size(k)
            pltpu.make_async_copy(
                bufs[k].at[pl.ds(0, sz)],
                out_hbm.at[base, pl.ds(0, sz)], osems[k]).wait()

    return emb_kernel


def kernel(x, lut):
    Bb, S = x.shape
    V, Dm = lut.shape
    return _make_kernel(Bb, S, V)(x.astype(jnp.int32), lut)
